# baseline (speedup 1.0000x reference)
import functools

import jax
import jax.numpy as jnp
from jax import lax
from jax.experimental import pallas as pl
from jax.experimental.pallas import tpu as pltpu
from jax.experimental.pallas import tpu_sc as plsc

_NUM_CORES = 1
_NUM_SUBCORES = 16
_NUM_WORKERS = _NUM_CORES * _NUM_SUBCORES
_LANES = 16


def kernel(authorIds, styles_table):
    (batch,) = authorIds.shape
    _, d = styles_table.shape
    b_per_w = batch // _NUM_WORKERS
    n_groups = b_per_w // _LANES

    mesh = plsc.VectorSubcoreMesh(
        core_axis_name="c", subcore_axis_name="s", num_cores=1
    )

    @functools.partial(
        pl.kernel,
        out_type=jax.ShapeDtypeStruct((batch, d), jnp.float32),
        mesh=mesh,
        scratch_types=[
            pltpu.VMEM((b_per_w,), jnp.int32),
            pltpu.VMEM((b_per_w // 2, d), jnp.float32),
            pltpu.SemaphoreType.DMA,
        ],
        compiler_params=pltpu.CompilerParams(
            use_tc_tiling_on_sc=True,
            skip_device_barrier=True,
            disable_bounds_checks=True,
            disable_semaphore_checks=True,
        ),
    )
    def gather_kernel(idx_hbm, table_hbm, out_hbm, idx_v, rows_v, sem):
        wid = lax.axis_index("s") * _NUM_CORES + lax.axis_index("c")
        base = wid * b_per_w
        pltpu.sync_copy(idx_hbm.at[pl.ds(base, b_per_w)], idx_v)

        half = b_per_w // 2
        n_groups_h = half // _LANES
        for h in range(2):
            def body(g, carry):
                v = idx_v[pl.ds(h * half + g * _LANES, _LANES)]
                for j in range(_LANES):
                    pltpu.async_copy(
                        table_hbm.at[pl.ds(v[j], 1)],
                        rows_v.at[pl.ds(g * _LANES + j, 1)],
                        sem,
                    )
                return carry

            lax.fori_loop(0, n_groups_h, body, 0)
            pltpu.make_async_copy(
                table_hbm.at[pl.ds(0, half)], rows_v, sem
            ).wait()

            pltpu.sync_copy(
                rows_v, out_hbm.at[pl.ds(base + h * half, half)]
            )

    return gather_kernel(authorIds.astype(jnp.int32), styles_table)


# speedup vs baseline: 1.3095x; 1.3095x over previous
import functools

import jax
import jax.numpy as jnp
from jax import lax
from jax.experimental import pallas as pl
from jax.experimental.pallas import tpu as pltpu
from jax.experimental.pallas import tpu_sc as plsc

_NUM_CORES = 2
_NUM_SUBCORES = 16
_NUM_WORKERS = _NUM_CORES * _NUM_SUBCORES
_LANES = 16
_OUT_CHUNK = 2048


def kernel(authorIds, styles_table):
    (batch,) = authorIds.shape
    vocab, d = styles_table.shape
    cols_per_w = d // _NUM_WORKERS

    mesh = plsc.VectorSubcoreMesh(core_axis_name="c", subcore_axis_name="s")

    @functools.partial(
        pl.kernel,
        out_type=jax.ShapeDtypeStruct((d, batch), jnp.float32),
        mesh=mesh,
        scratch_types=[
            pltpu.VMEM((batch,), jnp.int32),
            pltpu.VMEM((vocab,), jnp.float32),
            pltpu.VMEM((_OUT_CHUNK,), jnp.float32),
        ],
        compiler_params=pltpu.CompilerParams(
            use_tc_tiling_on_sc=True,
            needs_layout_passes=False,
            skip_device_barrier=True,
            disable_bounds_checks=True,
            disable_semaphore_checks=True,
        ),
    )
    def gather_kernel(idx_hbm, tableT_hbm, outT_hbm, idx_v, col_v, out_v):
        wid = lax.axis_index("s") * _NUM_CORES + lax.axis_index("c")
        pltpu.sync_copy(idx_hbm, idx_v)

        for p in range(cols_per_w):
            c = p * _NUM_WORKERS + wid
            pltpu.sync_copy(tableT_hbm.at[c], col_v)

            def chunk_body(k, carry):
                def group_body(g, carry2):
                    i16 = idx_v[pl.ds(k * _OUT_CHUNK + g * _LANES, _LANES)]
                    out_v[pl.ds(g * _LANES, _LANES)] = plsc.load_gather(
                        col_v, [i16]
                    )
                    return carry2

                lax.fori_loop(0, _OUT_CHUNK // _LANES, group_body, 0)
                pltpu.sync_copy(
                    out_v, outT_hbm.at[c, pl.ds(k * _OUT_CHUNK, _OUT_CHUNK)]
                )
                return carry

            lax.fori_loop(0, batch // _OUT_CHUNK, chunk_body, 0)

    outT = gather_kernel(authorIds.astype(jnp.int32), styles_table.T)
    return outT.T


# speedup vs baseline: 1.3430x; 1.0256x over previous
import functools

import jax
import jax.numpy as jnp
from jax import lax
from jax.experimental import pallas as pl
from jax.experimental.pallas import tpu as pltpu
from jax.experimental.pallas import tpu_sc as plsc

_NUM_CORES = 2
_NUM_SUBCORES = 16
_NUM_WORKERS = _NUM_CORES * _NUM_SUBCORES
_LANES = 16
_OUT_CHUNK = 2048


def kernel(authorIds, styles_table):
    (batch,) = authorIds.shape
    vocab, d = styles_table.shape
    cols_per_w = d // _NUM_WORKERS

    mesh = plsc.VectorSubcoreMesh(core_axis_name="c", subcore_axis_name="s")

    @functools.partial(
        pl.kernel,
        out_type=jax.ShapeDtypeStruct((d, batch), jnp.float32),
        mesh=mesh,
        scratch_types=[
            pltpu.VMEM((batch,), jnp.int32),
            pltpu.VMEM((vocab,), jnp.float32),
            pltpu.VMEM((_OUT_CHUNK,), jnp.float32),
        ],
        compiler_params=pltpu.CompilerParams(
            use_tc_tiling_on_sc=True,
            needs_layout_passes=False,
            skip_device_barrier=True,
            disable_bounds_checks=True,
            disable_semaphore_checks=True,
        ),
    )
    def gather_kernel(idx_hbm, tableT_hbm, outT_hbm, idx_v, col_v, out_v):
        wid = lax.axis_index("s") * _NUM_CORES + lax.axis_index("c")
        pltpu.sync_copy(idx_hbm, idx_v)

        for p in range(cols_per_w):
            c = p * _NUM_WORKERS + wid
            pltpu.sync_copy(tableT_hbm.at[c], col_v)

            def chunk_body(k, carry):
                def group_body(g, carry2):
                    i16 = idx_v[pl.ds(k * _OUT_CHUNK + g * _LANES, _LANES)]
                    out_v[pl.ds(g * _LANES, _LANES)] = plsc.load_gather(
                        col_v, [i16]
                    )
                    return carry2

                lax.fori_loop(0, _OUT_CHUNK // _LANES, group_body, 0, unroll=8)
                pltpu.sync_copy(
                    out_v, outT_hbm.at[c, pl.ds(k * _OUT_CHUNK, _OUT_CHUNK)]
                )
                return carry

            lax.fori_loop(0, batch // _OUT_CHUNK, chunk_body, 0)

    outT = gather_kernel(authorIds.astype(jnp.int32), styles_table.T)
    return outT.T


# speedup vs baseline: 1.3986x; 1.0414x over previous
import functools

import jax
import jax.numpy as jnp
from jax import lax
from jax.experimental import pallas as pl
from jax.experimental.pallas import tpu as pltpu
from jax.experimental.pallas import tpu_sc as plsc

_NUM_CORES = 2
_NUM_SUBCORES = 16
_NUM_WORKERS = _NUM_CORES * _NUM_SUBCORES
_LANES = 16
_OUT_CHUNK = 4096


def kernel(authorIds, styles_table):
    (batch,) = authorIds.shape
    vocab, d = styles_table.shape
    cols_per_w = d // _NUM_WORKERS
    n_chunks = batch // _OUT_CHUNK

    mesh = plsc.VectorSubcoreMesh(core_axis_name="c", subcore_axis_name="s")

    @functools.partial(
        pl.kernel,
        out_type=jax.ShapeDtypeStruct((d, batch), jnp.float32),
        mesh=mesh,
        scratch_types=[
            pltpu.VMEM((batch,), jnp.int32),
            pltpu.VMEM((vocab,), jnp.float32),
            pltpu.VMEM((_OUT_CHUNK,), jnp.float32),
            pltpu.VMEM((_OUT_CHUNK,), jnp.float32),
            pltpu.SemaphoreType.DMA,
            pltpu.SemaphoreType.DMA,
        ],
        compiler_params=pltpu.CompilerParams(
            use_tc_tiling_on_sc=True,
            needs_layout_passes=False,
            skip_device_barrier=True,
            disable_bounds_checks=True,
            disable_semaphore_checks=True,
        ),
    )
    def gather_kernel(
        idx_hbm, tableT_hbm, outT_hbm, idx_v, col_v, out_v0, out_v1, sem_col, sem_out
    ):
        wid = lax.axis_index("s") * _NUM_CORES + lax.axis_index("c")
        out_bufs = (out_v0, out_v1)

        first = pltpu.async_copy(
            tableT_hbm.at[0 * _NUM_WORKERS + wid], col_v, sem_col
        )
        pltpu.sync_copy(idx_hbm, idx_v)
        first.wait()

        pending = []
        for p in range(cols_per_w):
            c = p * _NUM_WORKERS + wid
            if p > 0:
                pltpu.sync_copy(tableT_hbm.at[c], col_v)

            for k in range(n_chunks):
                buf = out_bufs[(p * n_chunks + k) % 2]
                if len(pending) >= 2:
                    pending.pop(0).wait()

                def group_body(g, carry, _k=k, _buf=buf):
                    i16 = idx_v[pl.ds(_k * _OUT_CHUNK + g * _LANES, _LANES)]
                    _buf[pl.ds(g * _LANES, _LANES)] = plsc.load_gather(
                        col_v, [i16]
                    )
                    return carry

                lax.fori_loop(0, _OUT_CHUNK // _LANES, group_body, 0, unroll=8)
                pending.append(
                    pltpu.async_copy(
                        buf,
                        outT_hbm.at[c, pl.ds(k * _OUT_CHUNK, _OUT_CHUNK)],
                        sem_out,
                    )
                )
        for w in pending:
            w.wait()

    outT = gather_kernel(authorIds.astype(jnp.int32), styles_table.T)
    return outT.T


# speedup vs baseline: 1.9957x; 1.4269x over previous
import functools

import jax
import jax.numpy as jnp
from jax import lax
from jax.experimental import pallas as pl
from jax.experimental.pallas import tpu as pltpu
from jax.experimental.pallas import tpu_sc as plsc

_NUM_CORES = 2
_NUM_SUBCORES = 16
_NUM_WORKERS = _NUM_CORES * _NUM_SUBCORES
_LANES = 16
_OUT_CHUNK = 4096


def kernel(authorIds, styles_table):
    (batch,) = authorIds.shape
    vocab, d = styles_table.shape
    cols_per_w = d // _NUM_WORKERS
    n_chunks = batch // _OUT_CHUNK

    mesh = plsc.VectorSubcoreMesh(core_axis_name="c", subcore_axis_name="s")

    @functools.partial(
        pl.kernel,
        out_type=jax.ShapeDtypeStruct((d, batch), jnp.float32),
        mesh=mesh,
        scratch_types=[
            pltpu.VMEM((batch,), jnp.int32),
            pltpu.VMEM((vocab,), jnp.float32),
            pltpu.VMEM((_OUT_CHUNK,), jnp.float32),
            pltpu.VMEM((_OUT_CHUNK,), jnp.float32),
            pltpu.SemaphoreType.DMA,
            pltpu.SemaphoreType.DMA,
        ],
        compiler_params=pltpu.CompilerParams(
            use_tc_tiling_on_sc=True,
            needs_layout_passes=False,
            skip_device_barrier=True,
            disable_bounds_checks=True,
            disable_semaphore_checks=True,
        ),
    )
    def gather_kernel(
        idx_hbm, tableT_hbm, outT_hbm, idx_v, col_v, out_v0, out_v1, sem_col, sem_out
    ):
        wid = lax.axis_index("s") * _NUM_CORES + lax.axis_index("c")
        out_bufs = (out_v0, out_v1)

        first = pltpu.async_copy(
            tableT_hbm.at[0 * _NUM_WORKERS + wid], col_v, sem_col
        )
        pltpu.sync_copy(idx_hbm, idx_v)
        first.wait()

        pending = []
        for p in range(cols_per_w):
            c = p * _NUM_WORKERS + wid
            if p > 0:
                pltpu.sync_copy(tableT_hbm.at[c], col_v)

            for k in range(n_chunks):
                buf = out_bufs[(p * n_chunks + k) % 2]
                if len(pending) >= 2:
                    pending.pop(0).wait()

                @plsc.parallel_loop(0, _OUT_CHUNK, step=_LANES, unroll=8)
                def group_body(g, _k=k, _buf=buf):
                    i16 = idx_v[pl.ds(_k * _OUT_CHUNK + g, _LANES)]
                    _buf[pl.ds(g, _LANES)] = plsc.load_gather(col_v, [i16])
                pending.append(
                    pltpu.async_copy(
                        buf,
                        outT_hbm.at[c, pl.ds(k * _OUT_CHUNK, _OUT_CHUNK)],
                        sem_out,
                    )
                )
        for w in pending:
            w.wait()

    outT = gather_kernel(authorIds.astype(jnp.int32), styles_table.T)
    return outT.T
